# baseline (speedup 1.0000x reference)
import functools

import jax
import jax.numpy as jnp
from jax import lax
from jax.experimental import pallas as pl
from jax.experimental.pallas import tpu as pltpu
from jax.experimental.pallas import tpu_sc as plsc

VOCAB = 1000000
HID = 64
PAD = 0
SCALE = float(HID) ** 0.5

NC, NS, LANES = 2, 16, 16
NW = NC * NS
B_TOTAL = 4096 * 200
PER_W = B_TOTAL // NW
CHUNK = 128
NBUF = 4
N_CHUNKS = PER_W // CHUNK
assert N_CHUNKS % NBUF == 0


def _body(idx_hbm, table_hbm, out_hbm, idx_all,
          r0, r1, r2, r3, p0, p1, p2, p3,
          i0, i1, i2, i3, gs0, gs1, gs2, gs3, ss0, ss1, ss2, ss3):
  rbufs = [r0, r1, r2, r3]
  pbufs = [p0, p1, p2, p3]
  ibufs = [i0, i1, i2, i3]
  gsems = [gs0, gs1, gs2, gs3]
  ssems = [ss0, ss1, ss2, ss3]

  wid = lax.axis_index("s") * NC + lax.axis_index("c")
  base = wid * PER_W

  pltpu.sync_copy(idx_hbm.at[pl.ds(pl.multiple_of(base, 1024), PER_W)], idx_all)

  def start_gather(chunk, b):
    def halve(g, carry):
      v = idx_all[pl.ds(chunk * CHUNK + g * LANES, LANES)]
      ibufs[b][pl.ds(g * LANES, LANES)] = lax.shift_right_logical(v, 1)
      return carry
    lax.fori_loop(0, CHUNK // LANES, halve, 0)
    pltpu.async_copy(table_hbm.at[ibufs[b]], rbufs[b], gsems[b])

  dnums = lax.GatherDimensionNumbers(
      offset_dims=(), collapsed_slice_dims=(0,), start_index_map=(0,))

  def splat(vec, j):
    return lax.gather(vec, jnp.full((LANES, 1), j, jnp.int32), dnums, (1,),
                      mode=lax.GatherScatterMode.PROMISE_IN_BOUNDS)

  def scale_chunk(chunk, b):
    rows = rbufs[b]
    pack = pbufs[b]

    def group(g, carry):
      idxv = idx_all[pl.ds(chunk * CHUNK + g * LANES, LANES)]
      mv = jnp.minimum(idxv, 1).astype(jnp.float32) * SCALE
      parf = (idxv & 1).astype(jnp.float32)
      for j in range(LANES):
        mspl = splat(mv, j)
        pspl = splat(parf, j)
        r = g * LANES + j
        pr = g * (LANES // 2) + j // 2
        half = (j % 2) * HID
        for c in range(HID // LANES):
          lo = rows[r, pl.ds(c * LANES, LANES)]
          hi = rows[r, pl.ds(HID + c * LANES, LANES)]
          pack[pr, pl.ds(half + c * LANES, LANES)] = (
              lo + pspl * (hi - lo)) * mspl
      return carry

    lax.fori_loop(0, CHUNK // LANES, group, 0)

  for b in range(NBUF):
    start_gather(b, b)

  def outer(s, carry):
    for b in range(NBUF):
      chunk = s * NBUF + b
      pltpu.make_async_copy(
          table_hbm.at[ibufs[b]], rbufs[b], gsems[b]).wait()
      scale_chunk(chunk, b)
      off = pl.multiple_of((base + chunk * CHUNK) // 2, 64)
      pltpu.async_copy(
          pbufs[b],
          out_hbm.at[pl.ds(off, CHUNK // 2)], ssems[b])
      nxt = chunk + NBUF

      @pl.when(nxt < N_CHUNKS)
      def _():
        pltpu.make_async_copy(
            pbufs[b],
            out_hbm.at[pl.ds(off, CHUNK // 2)],
            ssems[b]).wait()
        start_gather(nxt, b)

    return carry

  lax.fori_loop(0, N_CHUNKS // NBUF, outer, 0)

  for b in range(NBUF):
    chunk = N_CHUNKS - NBUF + b
    off = pl.multiple_of((base + chunk * CHUNK) // 2, 64)
    pltpu.make_async_copy(
        pbufs[b],
        out_hbm.at[pl.ds(off, CHUNK // 2)],
        ssems[b]).wait()


@functools.partial(jax.jit, static_argnames=())
def _run(idx_flat, table2):
  mesh = plsc.VectorSubcoreMesh(core_axis_name="c", subcore_axis_name="s")
  k = pl.kernel(
      _body,
      out_type=jax.ShapeDtypeStruct((B_TOTAL // 2, 2 * HID), jnp.float32),
      mesh=mesh,
      scratch_types=(
          [pltpu.VMEM((PER_W,), jnp.int32)]
          + [pltpu.VMEM((CHUNK, 2 * HID), jnp.float32) for _ in range(NBUF)]
          + [pltpu.VMEM((CHUNK // 2, 2 * HID), jnp.float32)
             for _ in range(NBUF)]
          + [pltpu.VMEM((CHUNK,), jnp.int32) for _ in range(NBUF)]
          + [pltpu.SemaphoreType.DMA for _ in range(2 * NBUF)]
      ),
      compiler_params=pltpu.CompilerParams(use_tc_tiling_on_sc=True),
  )
  return k(idx_flat, table2)


def kernel(inputs, shared_weights):
  idx_flat = inputs.reshape(-1).astype(jnp.int32)
  table2 = shared_weights.reshape(VOCAB // 2, 2 * HID)
  out = _run(idx_flat, table2)
  return out.reshape(inputs.shape + (HID,))


# speedup vs baseline: 1.0120x; 1.0120x over previous
import functools

import jax
import jax.numpy as jnp
from jax import lax
from jax.experimental import pallas as pl
from jax.experimental.pallas import tpu as pltpu
from jax.experimental.pallas import tpu_sc as plsc

VOCAB = 1000000
HID = 64
PAD = 0
SCALE = float(HID) ** 0.5

NC, NS, LANES = 2, 16, 16
NW = NC * NS
NB, NT = 4096, 200
BBLK = NB // NW
NBUF = 4
assert NT % NBUF == 0


def _body(idxt_hbm, table_hbm, out_hbm, idx_all,
          r0, r1, r2, r3, o0, o1, o2, o3,
          i0, i1, i2, i3, gs0, gs1, gs2, gs3, ss0, ss1, ss2, ss3):
  rbufs = [r0, r1, r2, r3]
  obufs = [o0, o1, o2, o3]
  ibufs = [i0, i1, i2, i3]
  gsems = [gs0, gs1, gs2, gs3]
  ssems = [ss0, ss1, ss2, ss3]

  wid = lax.axis_index("s") * NC + lax.axis_index("c")
  bbase = pl.multiple_of(wid * BBLK, BBLK)

  pltpu.sync_copy(idxt_hbm.at[:, pl.ds(bbase, BBLK)], idx_all)

  def start_gather(t, b):
    def halve(g, carry):
      v = idx_all[t, pl.ds(g * LANES, LANES)]
      ibufs[b][pl.ds(g * LANES, LANES)] = lax.shift_right_logical(v, 1)
      return carry
    lax.fori_loop(0, BBLK // LANES, halve, 0)
    pltpu.async_copy(table_hbm.at[ibufs[b]], rbufs[b], gsems[b])

  lane_iota = lax.iota(jnp.int32, LANES)

  def assemble(t, b):
    rows = rbufs[b]
    out = obufs[b]

    def group(g, carry):
      idxv = idx_all[t, pl.ds(g * LANES, LANES)]
      mv = jnp.minimum(idxv, 1).astype(jnp.float32) * SCALE
      rowids = g * LANES + lane_iota
      half = (idxv & 1) * HID
      for h in range(HID):
        v = plsc.load_gather(rows, [rowids, half + h])
        out[h, pl.ds(g * LANES, LANES)] = v * mv
      return carry

    lax.fori_loop(0, BBLK // LANES, group, 0)

  for b in range(NBUF):
    start_gather(b, b)

  def outer(s, carry):
    for b in range(NBUF):
      t = s * NBUF + b
      pltpu.make_async_copy(
          table_hbm.at[ibufs[b]], rbufs[b], gsems[b]).wait()
      assemble(t, b)
      pltpu.async_copy(
          obufs[b], out_hbm.at[t, :, pl.ds(bbase, BBLK)], ssems[b])
      nxt = t + NBUF

      @pl.when(nxt < NT)
      def _():
        pltpu.make_async_copy(
            obufs[b], out_hbm.at[t, :, pl.ds(bbase, BBLK)], ssems[b]).wait()
        start_gather(nxt, b)

    return carry

  lax.fori_loop(0, NT // NBUF, outer, 0)

  for b in range(NBUF):
    t = NT - NBUF + b
    pltpu.make_async_copy(
        obufs[b], out_hbm.at[t, :, pl.ds(bbase, BBLK)], ssems[b]).wait()


@functools.partial(jax.jit, static_argnames=())
def _run(idxt, table2):
  mesh = plsc.VectorSubcoreMesh(core_axis_name="c", subcore_axis_name="s")
  k = pl.kernel(
      _body,
      out_type=jax.ShapeDtypeStruct((NT, HID, NB), jnp.float32),
      mesh=mesh,
      scratch_types=(
          [pltpu.VMEM((NT, BBLK), jnp.int32)]
          + [pltpu.VMEM((BBLK, 2 * HID), jnp.float32) for _ in range(NBUF)]
          + [pltpu.VMEM((HID, BBLK), jnp.float32) for _ in range(NBUF)]
          + [pltpu.VMEM((BBLK,), jnp.int32) for _ in range(NBUF)]
          + [pltpu.SemaphoreType.DMA for _ in range(2 * NBUF)]
      ),
      compiler_params=pltpu.CompilerParams(use_tc_tiling_on_sc=True,
                                           needs_layout_passes=False),
  )
  return k(idxt, table2)


def kernel(inputs, shared_weights):
  idxt = inputs.T.astype(jnp.int32)
  table2 = shared_weights.reshape(VOCAB // 2, 2 * HID)
  out3 = _run(idxt, table2)
  return out3.transpose(2, 0, 1)


# speedup vs baseline: 1.0414x; 1.0290x over previous
import functools

import jax
import jax.numpy as jnp
from jax import lax
from jax.experimental import pallas as pl
from jax.experimental.pallas import tpu as pltpu
from jax.experimental.pallas import tpu_sc as plsc

VOCAB = 1000000
HID = 64
PAD = 0
SCALE = float(HID) ** 0.5

NC, NS, LANES = 2, 16, 16
NW = NC * NS
NB, NT = 4096, 200
BBLK = NB // NW
RPAD = 2 * HID + 1
NBUF = 2
assert NT % NBUF == 0


def _body(idxt_hbm, table_hbm, out_hbm, idx_all,
          r0, r1, o0, o1, i0, i1, gs0, gs1, ss0, ss1):
  rbufs = [r0, r1]
  obufs = [o0, o1]
  ibufs = [i0, i1]
  gsems = [gs0, gs1]
  ssems = [ss0, ss1]

  wid = lax.axis_index("s") * NC + lax.axis_index("c")
  bbase = pl.multiple_of(wid * BBLK, BBLK)

  pltpu.sync_copy(idxt_hbm.at[:, pl.ds(bbase, BBLK)], idx_all)

  def start_gather(t, b):
    def halve(g, carry):
      v = idx_all[t, pl.ds(g * LANES, LANES)]
      ibufs[b][pl.ds(g * LANES, LANES)] = lax.shift_right_logical(v, 1)
      return carry
    lax.fori_loop(0, BBLK // LANES, halve, 0)
    pltpu.async_copy(table_hbm.at[ibufs[b]],
                     rbufs[b].at[:, pl.ds(0, 2 * HID)], gsems[b])

  lane_iota = lax.iota(jnp.int32, LANES)

  def assemble(t, b):
    rows = rbufs[b]
    out = obufs[b]

    def group(g, carry):
      idxv = idx_all[t, pl.ds(g * LANES, LANES)]
      mv = jnp.minimum(idxv, 1).astype(jnp.float32) * SCALE
      rowids = g * LANES + lane_iota
      half = (idxv & 1) * HID
      for h in range(HID):
        v = plsc.load_gather(rows, [rowids, half + h])
        out[h, pl.ds(g * LANES, LANES)] = v * mv
      return carry

    lax.fori_loop(0, BBLK // LANES, group, 0)

  for b in range(NBUF):
    start_gather(b, b)

  def outer(s, carry):
    for b in range(NBUF):
      t = s * NBUF + b
      pltpu.make_async_copy(
          table_hbm.at[ibufs[b]],
          rbufs[b].at[:, pl.ds(0, 2 * HID)], gsems[b]).wait()

      @pl.when(t >= NBUF)
      def _():
        pltpu.make_async_copy(
            obufs[b], out_hbm.at[t - NBUF, :, pl.ds(bbase, BBLK)],
            ssems[b]).wait()

      assemble(t, b)
      pltpu.async_copy(
          obufs[b], out_hbm.at[t, :, pl.ds(bbase, BBLK)], ssems[b])

      @pl.when(t + NBUF < NT)
      def _():
        start_gather(t + NBUF, b)

    return carry

  lax.fori_loop(0, NT // NBUF, outer, 0)

  for b in range(NBUF):
    t = NT - NBUF + b
    pltpu.make_async_copy(
        obufs[b], out_hbm.at[t, :, pl.ds(bbase, BBLK)], ssems[b]).wait()


@functools.partial(jax.jit, static_argnames=())
def _run(idxt, table2):
  mesh = plsc.VectorSubcoreMesh(core_axis_name="c", subcore_axis_name="s")
  k = pl.kernel(
      _body,
      out_type=jax.ShapeDtypeStruct((NT, HID, NB), jnp.float32),
      mesh=mesh,
      scratch_types=(
          [pltpu.VMEM((NT, BBLK), jnp.int32)]
          + [pltpu.VMEM((BBLK, RPAD), jnp.float32) for _ in range(NBUF)]
          + [pltpu.VMEM((HID, BBLK), jnp.float32) for _ in range(NBUF)]
          + [pltpu.VMEM((BBLK,), jnp.int32) for _ in range(NBUF)]
          + [pltpu.SemaphoreType.DMA for _ in range(2 * NBUF)]
      ),
      compiler_params=pltpu.CompilerParams(use_tc_tiling_on_sc=True,
                                           needs_layout_passes=False),
  )
  return k(idxt, table2)


def kernel(inputs, shared_weights):
  idxt = inputs.T.astype(jnp.int32)
  table2 = shared_weights.reshape(VOCAB // 2, 2 * HID)
  out3 = _run(idxt, table2)
  return out3.transpose(2, 0, 1)


# speedup vs baseline: 1.6618x; 1.5958x over previous
import functools

import jax
import jax.numpy as jnp
from jax import lax
from jax.experimental import pallas as pl
from jax.experimental.pallas import tpu as pltpu
from jax.experimental.pallas import tpu_sc as plsc

VOCAB = 1000000
HID = 64
PAD = 0
SCALE = float(HID) ** 0.5

NC, NS, LANES = 2, 16, 16
NW = NC * NS
B_TOTAL = 4096 * 200
PER_W = B_TOTAL // NW
CHUNK = 320
NBUF = 2
N_CHUNKS = PER_W // CHUNK
assert N_CHUNKS % NBUF == 0


def _body(idx_hbm, table_hbm, out_hbm, idx_all,
          r0, r1, o0, o1, gs0, gs1, ss0, ss1):
  rbufs = [r0, r1]
  obufs = [o0, o1]
  gsems = [gs0, gs1]
  ssems = [ss0, ss1]

  wid = lax.axis_index("s") * NC + lax.axis_index("c")
  base = pl.multiple_of(wid * PER_W, 1024)

  pltpu.sync_copy(idx_hbm.at[pl.ds(base, PER_W)], idx_all)

  def start_gather(chunk, b):
    pltpu.async_copy(
        table_hbm.at[idx_all.at[pl.ds(chunk * CHUNK, CHUNK)]],
        rbufs[b], gsems[b])

  dnums = lax.GatherDimensionNumbers(
      offset_dims=(), collapsed_slice_dims=(0,), start_index_map=(0,))

  def splat(vec, j):
    return lax.gather(vec, jnp.full((LANES, 1), j, jnp.int32), dnums, (1,),
                      mode=lax.GatherScatterMode.PROMISE_IN_BOUNDS)

  def scale_chunk(chunk, b):
    rows = rbufs[b]
    dst = obufs[b]

    def group(g, carry):
      idxv = idx_all[pl.ds(chunk * CHUNK + g * LANES, LANES)]
      mv = jnp.minimum(idxv, 1).astype(jnp.float32) * SCALE
      for j in range(LANES):
        mspl = splat(mv, j)
        r = g * LANES + j
        for c in range(HID // LANES):
          dst[r, pl.ds(c * LANES, LANES)] = (
              rows[r, pl.ds(c * LANES, LANES)] * mspl)
      return carry

    lax.fori_loop(0, CHUNK // LANES, group, 0)

  for b in range(NBUF):
    start_gather(b, b)

  def outer(s, carry):
    for b in range(NBUF):
      chunk = s * NBUF + b
      pltpu.make_async_copy(
          table_hbm.at[idx_all.at[pl.ds(chunk * CHUNK, CHUNK)]],
          rbufs[b], gsems[b]).wait()

      @pl.when(chunk >= NBUF)
      def _():
        off0 = pl.multiple_of(base + (chunk - NBUF) * CHUNK, 64)
        pltpu.make_async_copy(
            obufs[b], out_hbm.at[pl.ds(off0, CHUNK)], ssems[b]).wait()

      scale_chunk(chunk, b)
      off = pl.multiple_of(base + chunk * CHUNK, 64)
      pltpu.async_copy(obufs[b], out_hbm.at[pl.ds(off, CHUNK)], ssems[b])

      @pl.when(chunk + NBUF < N_CHUNKS)
      def _():
        start_gather(chunk + NBUF, b)

    return carry

  lax.fori_loop(0, N_CHUNKS // NBUF, outer, 0)

  for b in range(NBUF):
    chunk = N_CHUNKS - NBUF + b
    off = pl.multiple_of(base + chunk * CHUNK, 64)
    pltpu.make_async_copy(
        obufs[b], out_hbm.at[pl.ds(off, CHUNK)], ssems[b]).wait()


@functools.partial(jax.jit, static_argnames=())
def _run(idx_flat, table):
  mesh = plsc.VectorSubcoreMesh(core_axis_name="c", subcore_axis_name="s")
  k = pl.kernel(
      _body,
      out_type=jax.ShapeDtypeStruct((B_TOTAL, HID), jnp.float32),
      mesh=mesh,
      scratch_types=(
          [pltpu.VMEM((PER_W,), jnp.int32)]
          + [pltpu.VMEM((CHUNK, HID), jnp.float32) for _ in range(NBUF)]
          + [pltpu.VMEM((CHUNK, HID), jnp.float32) for _ in range(NBUF)]
          + [pltpu.SemaphoreType.DMA for _ in range(2 * NBUF)]
      ),
      compiler_params=pltpu.CompilerParams(use_tc_tiling_on_sc=False),
  )
  return k(idx_flat, table)


def kernel(inputs, shared_weights):
  idx_flat = inputs.reshape(-1).astype(jnp.int32)
  out = _run(idx_flat, shared_weights)
  return out.reshape(inputs.shape + (HID,))
